# baseline (speedup 1.0000x reference)
import functools

import jax
import jax.numpy as jnp
from jax import lax
from jax.experimental import pallas as pl
from jax.experimental.pallas import tpu as pltpu
from jax.experimental.pallas import tpu_sc as plsc

N = 100000
C = 128
L = 16
NW = 32
G = N // L
PG = -(-G // NW)
CG = 14
CH = CG * L
NCH = PG // CG


def _body(x_hbm, idx_hbm, out_hbm, x_v, out_v, idx_v, t64_v, sq_v):
    wid = lax.axis_index("s") * 2 + lax.axis_index("c")
    tstart = jnp.minimum(wid * PG, G - PG) * L

    iota16 = lax.iota(jnp.int32, L)

    pltpu.sync_copy(x_hbm.at[:, pl.ds(0, 64)], t64_v)

    def sq_body(e, carry):
        ev = jnp.full((L,), e, jnp.int32)
        for cg in range(C // L):
            col = plsc.load_gather(t64_v, [iota16 + cg * L, ev])
            sq_v[e, pl.ds(cg * L, L)] = col * col
        return carry

    lax.fori_loop(0, 64, sq_body, 0, unroll=4)

    def chunk_body(k, carry):
        rbase = tstart + k * CH
        pltpu.sync_copy(idx_hbm.at[pl.ds(rbase, CH)], idx_v)
        pltpu.sync_copy(x_hbm.at[:, pl.ds(rbase, CH)], x_v)

        def group_body(ig, c2):
            ev = jnp.clip(idx_v[pl.ds(ig * L, L)], 0, 63)
            rvec = ig * L + iota16

            def c_body(c, c3):
                cv = jnp.full((L,), c, jnp.int32)
                xv = x_v[c, pl.ds(ig * L, L)]
                sv = plsc.load_gather(sq_v, [ev, cv])
                plsc.store_scatter(out_v, [rvec, cv], xv * sv)
                return c3

            lax.fori_loop(0, C, c_body, 0, unroll=16)
            return c2

        lax.fori_loop(0, CG, group_body, 0)
        pltpu.sync_copy(out_v, out_hbm.at[pl.ds(rbase, CH), :])
        return carry

    lax.fori_loop(0, NCH, chunk_body, 0)


@jax.jit
def kernel(inp1, inp2):
    idx32 = inp2.reshape(-1).astype(jnp.int32)
    mesh = plsc.VectorSubcoreMesh(core_axis_name="c", subcore_axis_name="s")
    run = functools.partial(
        pl.kernel,
        mesh=mesh,
        compiler_params=pltpu.CompilerParams(
            use_tc_tiling_on_sc=False, needs_layout_passes=False
        ),
        out_type=jax.ShapeDtypeStruct((N, C), jnp.float32),
        scratch_types=[
            pltpu.VMEM((C, CH), jnp.float32),
            pltpu.VMEM((CH, C), jnp.float32),
            pltpu.VMEM((CH,), jnp.int32),
            pltpu.VMEM((C, 64), jnp.float32),
            pltpu.VMEM((64, C), jnp.float32),
        ],
    )(_body)
    return run(inp1, idx32)


# speedup vs baseline: 1.8766x; 1.8766x over previous
import functools

import jax
import jax.numpy as jnp
from jax import lax
from jax.experimental import pallas as pl
from jax.experimental.pallas import tpu as pltpu
from jax.experimental.pallas import tpu_sc as plsc

N = 100000
C = 128
L = 16
NW = 32
G = N // L
PG = -(-G // NW)
CG = 14
CH = CG * L
NCH = PG // CG


def _body(x_hbm, idx_hbm, out_hbm, x_v, out_v, idx_v, t64_v, sq_v):
    wid = lax.axis_index("s") * 2 + lax.axis_index("c")
    tstart = jnp.minimum(wid * PG, G - PG) * L

    iota16 = lax.iota(jnp.int32, L)

    pltpu.sync_copy(x_hbm.at[:, pl.ds(0, 64)], t64_v)

    def sq_body(e, carry):
        ev = jnp.full((L,), e, jnp.int32)
        for cg in range(C // L):
            col = plsc.load_gather(t64_v, [iota16 + cg * L, ev])
            sq_v[e, pl.ds(cg * L, L)] = col * col
        return carry

    lax.fori_loop(0, 64, sq_body, 0, unroll=4)

    def chunk_body(k, carry):
        rbase = tstart + k * CH
        pltpu.sync_copy(idx_hbm.at[pl.ds(rbase, CH)], idx_v)
        pltpu.sync_copy(x_hbm.at[:, pl.ds(rbase, CH)], x_v.at[:, pl.ds(0, CH)])

        def group_body(ig, c2):
            ev = jnp.clip(idx_v[pl.ds(ig * L, L)], 0, 63)
            for j in range(L):
                e = ev[j]
                i = ig * L + j
                iv = jnp.full((L,), i, jnp.int32)
                for cg in range(C // L):
                    xv = plsc.load_gather(x_v, [cg * L + iota16, iv])
                    sv = sq_v[e, pl.ds(cg * L, L)]
                    out_v[i, pl.ds(cg * L, L)] = xv * sv
            return c2

        lax.fori_loop(0, CG, group_body, 0)
        pltpu.sync_copy(out_v, out_hbm.at[pl.ds(rbase, CH), :])
        return carry

    lax.fori_loop(0, NCH, chunk_body, 0)


@jax.jit
def kernel(inp1, inp2):
    idx32 = inp2.reshape(-1).astype(jnp.int32)
    mesh = plsc.VectorSubcoreMesh(core_axis_name="c", subcore_axis_name="s")
    run = functools.partial(
        pl.kernel,
        mesh=mesh,
        compiler_params=pltpu.CompilerParams(
            use_tc_tiling_on_sc=False, needs_layout_passes=False
        ),
        out_type=jax.ShapeDtypeStruct((N, C), jnp.float32),
        scratch_types=[
            pltpu.VMEM((C, CH + 1), jnp.float32),
            pltpu.VMEM((CH, C), jnp.float32),
            pltpu.VMEM((CH,), jnp.int32),
            pltpu.VMEM((C, 64), jnp.float32),
            pltpu.VMEM((64, C), jnp.float32),
        ],
    )(_body)
    return run(inp1, idx32)


# speedup vs baseline: 2.0949x; 1.1163x over previous
import functools

import jax
import jax.numpy as jnp
from jax import lax
from jax.experimental import pallas as pl
from jax.experimental.pallas import tpu as pltpu
from jax.experimental.pallas import tpu_sc as plsc

N = 100000
C = 128
L = 16
NW = 32
G = N // L
PG = -(-G // NW)
CG = 14
CH = CG * L
CHP = CH + 1
NCH = PG // CG


def _body(x_hbm, idx_hbm, out_hbm,
          x0, x1, o0, o1, i0, i1, sq_v,
          sx0, sx1, si0, si1, so0, so1):
    wid = lax.axis_index("s") * 2 + lax.axis_index("c")
    tstart = jnp.minimum(wid * PG, G - PG) * L

    iota16 = lax.iota(jnp.int32, L)
    xbufs = (x0, x1)
    obufs = (o0, o1)
    ibufs = (i0, i1)
    sxs = (sx0, sx1)
    sis = (si0, si1)
    sos = (so0, so1)

    pltpu.sync_copy(x_hbm.at[:, pl.ds(0, 64)], x0.at[:, pl.ds(0, 64)])

    def sq_body(e, carry):
        ev = jnp.full((L,), e, jnp.int32)
        for cg in range(C // L):
            col = plsc.load_gather(x0, [iota16 + cg * L, ev])
            sq_v[e, pl.ds(cg * L, L)] = col * col
        return carry

    lax.fori_loop(0, 64, sq_body, 0, unroll=4)

    def start_fetch(k, p):
        rbase = tstart + k * CH
        cpx = pltpu.make_async_copy(
            x_hbm.at[:, pl.ds(rbase, CH)], xbufs[p].at[:, pl.ds(0, CH)],
            sxs[p])
        cpx.start()
        cpi = pltpu.make_async_copy(
            idx_hbm.at[pl.ds(rbase, CH)], ibufs[p], sis[p])
        cpi.start()

    def wait_fetch(k, p):
        rbase = tstart + k * CH
        pltpu.make_async_copy(
            x_hbm.at[:, pl.ds(rbase, CH)], xbufs[p].at[:, pl.ds(0, CH)],
            sxs[p]).wait()
        pltpu.make_async_copy(
            idx_hbm.at[pl.ds(rbase, CH)], ibufs[p], sis[p]).wait()

    def out_copy(k, p):
        rbase = tstart + k * CH
        return pltpu.make_async_copy(
            obufs[p], out_hbm.at[pl.ds(rbase, CH), :], sos[p])

    def compute(p):
        x_v, out_v, idx_v = xbufs[p], obufs[p], ibufs[p]

        def group_body(ig, c2):
            ev = jnp.clip(idx_v[pl.ds(ig * L, L)], 0, 63)
            for j in range(L):
                e = ev[j]
                i = ig * L + j
                iv = jnp.full((L,), i, jnp.int32)
                for cg in range(C // L):
                    xv = plsc.load_gather(x_v, [cg * L + iota16, iv])
                    sv = sq_v[e, pl.ds(cg * L, L)]
                    out_v[i, pl.ds(cg * L, L)] = xv * sv
            return c2

        lax.fori_loop(0, CG, group_body, 0)

    start_fetch(0, 0)
    start_fetch(1, 1)

    def pair_body(kk, carry):
        for p in range(2):
            k = 2 * kk + p
            wait_fetch(k, p)

            @pl.when(kk >= 1)
            def _():
                out_copy(k - 2, p).wait()

            compute(p)
            out_copy(k, p).start()

            @pl.when(kk < NCH // 2 - 1)
            def _():
                start_fetch(k + 2, p)
        return carry

    lax.fori_loop(0, NCH // 2, pair_body, 0)
    out_copy(NCH - 2, 0).wait()
    out_copy(NCH - 1, 1).wait()


@jax.jit
def kernel(inp1, inp2):
    idx32 = inp2.reshape(-1).astype(jnp.int32)
    mesh = plsc.VectorSubcoreMesh(core_axis_name="c", subcore_axis_name="s")
    run = functools.partial(
        pl.kernel,
        mesh=mesh,
        compiler_params=pltpu.CompilerParams(
            use_tc_tiling_on_sc=False, needs_layout_passes=False
        ),
        out_type=jax.ShapeDtypeStruct((N, C), jnp.float32),
        scratch_types=[
            pltpu.VMEM((C, CHP), jnp.float32),
            pltpu.VMEM((C, CHP), jnp.float32),
            pltpu.VMEM((CH, C), jnp.float32),
            pltpu.VMEM((CH, C), jnp.float32),
            pltpu.VMEM((CH,), jnp.int32),
            pltpu.VMEM((CH,), jnp.int32),
            pltpu.VMEM((64, C), jnp.float32),
            pltpu.SemaphoreType.DMA,
            pltpu.SemaphoreType.DMA,
            pltpu.SemaphoreType.DMA,
            pltpu.SemaphoreType.DMA,
            pltpu.SemaphoreType.DMA,
            pltpu.SemaphoreType.DMA,
        ],
    )(_body)
    return run(inp1, idx32)


# speedup vs baseline: 2.8030x; 1.3380x over previous
import functools

import jax
import jax.numpy as jnp
from jax import lax
from jax.experimental import pallas as pl
from jax.experimental.pallas import tpu as pltpu
from jax.experimental.pallas import tpu_sc as plsc

N = 100000
C = 128
L = 16
NW = 32
G = N // L
PG = -(-G // NW)
CG = 14
CH = CG * L
CHP = CH + 1
NCH = PG // CG


def _body(x_hbm, idx_hbm, out_hbm,
          x0, x1, o0, o1, i0, i1, sq_v,
          sx0, sx1, si0, si1, so0, so1):
    wid = lax.axis_index("s") * 2 + lax.axis_index("c")
    tstart = jnp.minimum(wid * PG, G - PG) * L

    iota16 = lax.iota(jnp.int32, L)
    xbufs = (x0, x1)
    obufs = (o0, o1)
    ibufs = (i0, i1)
    sxs = (sx0, sx1)
    sis = (si0, si1)
    sos = (so0, so1)

    pltpu.sync_copy(x_hbm.at[:, pl.ds(0, 64)], x0.at[:, pl.ds(0, 64)])

    @plsc.parallel_loop(0, 64, unroll=2)
    def sq_body(e):
        ev = jnp.full((L,), e, jnp.int32)
        for cg in range(C // L):
            col = plsc.load_gather(x0, [iota16 + cg * L, ev])
            sq_v[e, pl.ds(cg * L, L)] = col * col

    def start_fetch(k, p):
        rbase = tstart + k * CH
        cpx = pltpu.make_async_copy(
            x_hbm.at[:, pl.ds(rbase, CH)], xbufs[p].at[:, pl.ds(0, CH)],
            sxs[p])
        cpx.start()
        cpi = pltpu.make_async_copy(
            idx_hbm.at[pl.ds(rbase, CH)], ibufs[p], sis[p])
        cpi.start()

    def wait_fetch(k, p):
        rbase = tstart + k * CH
        pltpu.make_async_copy(
            x_hbm.at[:, pl.ds(rbase, CH)], xbufs[p].at[:, pl.ds(0, CH)],
            sxs[p]).wait()
        pltpu.make_async_copy(
            idx_hbm.at[pl.ds(rbase, CH)], ibufs[p], sis[p]).wait()

    def out_copy(k, p):
        rbase = tstart + k * CH
        return pltpu.make_async_copy(
            obufs[p], out_hbm.at[pl.ds(rbase, CH), :], sos[p])

    def compute(p):
        x_v, out_v, idx_v = xbufs[p], obufs[p], ibufs[p]

        @plsc.parallel_loop(0, CG)
        def group_body(ig):
            ev = jnp.clip(idx_v[pl.ds(ig * L, L)], 0, 63)
            for j in range(L):
                e = ev[j]
                i = ig * L + j
                iv = jnp.full((L,), i, jnp.int32)
                for cg in range(C // L):
                    xv = plsc.load_gather(x_v, [cg * L + iota16, iv])
                    sv = sq_v[e, pl.ds(cg * L, L)]
                    out_v[i, pl.ds(cg * L, L)] = xv * sv

    start_fetch(0, 0)
    start_fetch(1, 1)

    def pair_body(kk, carry):
        for p in range(2):
            k = 2 * kk + p
            wait_fetch(k, p)

            @pl.when(kk >= 1)
            def _():
                out_copy(k - 2, p).wait()

            compute(p)
            out_copy(k, p).start()

            @pl.when(kk < NCH // 2 - 1)
            def _():
                start_fetch(k + 2, p)
        return carry

    lax.fori_loop(0, NCH // 2, pair_body, 0)
    out_copy(NCH - 2, 0).wait()
    out_copy(NCH - 1, 1).wait()


@jax.jit
def kernel(inp1, inp2):
    idx32 = inp2.reshape(-1).astype(jnp.int32)
    mesh = plsc.VectorSubcoreMesh(core_axis_name="c", subcore_axis_name="s")
    run = functools.partial(
        pl.kernel,
        mesh=mesh,
        compiler_params=pltpu.CompilerParams(
            use_tc_tiling_on_sc=False, needs_layout_passes=False
        ),
        out_type=jax.ShapeDtypeStruct((N, C), jnp.float32),
        scratch_types=[
            pltpu.VMEM((C, CHP), jnp.float32),
            pltpu.VMEM((C, CHP), jnp.float32),
            pltpu.VMEM((CH, C), jnp.float32),
            pltpu.VMEM((CH, C), jnp.float32),
            pltpu.VMEM((CH,), jnp.int32),
            pltpu.VMEM((CH,), jnp.int32),
            pltpu.VMEM((64, C), jnp.float32),
            pltpu.SemaphoreType.DMA,
            pltpu.SemaphoreType.DMA,
            pltpu.SemaphoreType.DMA,
            pltpu.SemaphoreType.DMA,
            pltpu.SemaphoreType.DMA,
            pltpu.SemaphoreType.DMA,
        ],
    )(_body)
    return run(inp1, idx32)


# speedup vs baseline: 5.8094x; 2.0726x over previous
import functools

import jax
import jax.numpy as jnp
from jax import lax
from jax.experimental import pallas as pl
from jax.experimental.pallas import tpu as pltpu
from jax.experimental.pallas import tpu_sc as plsc

N = 100000
C = 128
L = 16
NW = 32
CH = 128
CG = CH // L
NCH = 25
SPAN = NCH * CH
TBASE = N - CH


def _body(x_hbm, idx_hbm, xt_hbm, it_hbm, out_hbm,
          x0, x1, o0, o1, i0, i1, sq_v,
          sx0, sx1, si0, si1, so0, so1):
    wid = lax.axis_index("s") * 2 + lax.axis_index("c")
    tstart = jnp.minimum(wid * SPAN, ((N - SPAN) // CH) * CH)

    iota16 = lax.iota(jnp.int32, L)
    xbufs = (x0, x1)
    obufs = (o0, o1)
    ibufs = (i0, i1)
    sxs = (sx0, sx1)
    sis = (si0, si1)
    sos = (so0, so1)

    pltpu.sync_copy(x_hbm.at[:, pl.ds(0, CH)], x0)

    @plsc.parallel_loop(0, 64)
    def sq_body(q):
        evec = (q & ~(L - 1)) + jnp.bitwise_and(iota16 + q, L - 1)
        for cg in range(C // L):
            cvec = cg * L + iota16
            col = plsc.load_gather(x0, [cvec, evec])
            plsc.store_scatter(sq_v, [evec, cvec], col * col)

    def start_fetch(k, p):
        rbase = tstart + k * CH
        pltpu.make_async_copy(
            x_hbm.at[:, pl.ds(rbase, CH)], xbufs[p], sxs[p]).start()
        pltpu.make_async_copy(
            idx_hbm.at[pl.ds(rbase, CH)], ibufs[p], sis[p]).start()

    def wait_fetch(k, p):
        rbase = tstart + k * CH
        pltpu.make_async_copy(
            x_hbm.at[:, pl.ds(rbase, CH)], xbufs[p], sxs[p]).wait()
        pltpu.make_async_copy(
            idx_hbm.at[pl.ds(rbase, CH)], ibufs[p], sis[p]).wait()

    def out_copy_to(rbase, p):
        return pltpu.make_async_copy(
            obufs[p], out_hbm.at[pl.ds(rbase, CH), :], sos[p])

    def out_copy(k, p):
        return out_copy_to(tstart + k * CH, p)

    def compute(p):
        x_v, out_v, idx_v = xbufs[p], obufs[p], ibufs[p]

        @plsc.parallel_loop(0, CH)
        def group_body(q):
            rvec = (q & ~(L - 1)) + jnp.bitwise_and(iota16 + q, L - 1)
            ev_rot = jnp.clip(plsc.load_gather(idx_v, [rvec]), 0, 63)
            for cg in range(C // L):
                cvec = cg * L + iota16
                xv = plsc.load_gather(x_v, [cvec, rvec])
                sv = plsc.load_gather(sq_v, [ev_rot, cvec])
                plsc.store_scatter(out_v, [rvec, cvec], xv * sv)

    start_fetch(0, 0)
    start_fetch(1, 1)

    def pair_body(kk, carry):
        for p in range(2):
            k = 2 * kk + p
            wait_fetch(k, p)

            @pl.when(kk >= 1)
            def _():
                out_copy(k - 2, p).wait()

            compute(p)
            out_copy(k, p).start()

            @pl.when(k + 2 < NCH)
            def _():
                start_fetch(k + 2, p)
        return carry

    lax.fori_loop(0, NCH // 2, pair_body, 0)

    k = NCH - 1
    wait_fetch(k, 0)
    out_copy(k - 2, 0).wait()
    compute(0)
    out_copy(k, 0).start()
    out_copy(k - 1, 1).wait()
    out_copy(k, 0).wait()

    @pl.when(wid == NW - 1)
    def _tail():
        pltpu.sync_copy(xt_hbm, x1)
        pltpu.sync_copy(it_hbm, i1)
        compute(1)
        cp = out_copy_to(TBASE, 1)
        cp.start()
        cp.wait()


@jax.jit
def kernel(inp1, inp2):
    idx32 = inp2.reshape(-1).astype(jnp.int32)
    tail_x = lax.slice(inp1, (0, TBASE), (C, N))
    tail_i = lax.slice(idx32, (TBASE,), (N,))
    mesh = plsc.VectorSubcoreMesh(core_axis_name="c", subcore_axis_name="s")
    run = functools.partial(
        pl.kernel,
        mesh=mesh,
        compiler_params=pltpu.CompilerParams(
            use_tc_tiling_on_sc=True, needs_layout_passes=False
        ),
        out_type=jax.ShapeDtypeStruct((N, C), jnp.float32),
        scratch_types=[
            pltpu.VMEM((C, CH), jnp.float32),
            pltpu.VMEM((C, CH), jnp.float32),
            pltpu.VMEM((CH, C), jnp.float32),
            pltpu.VMEM((CH, C), jnp.float32),
            pltpu.VMEM((CH,), jnp.int32),
            pltpu.VMEM((CH,), jnp.int32),
            pltpu.VMEM((64, C), jnp.float32),
            pltpu.SemaphoreType.DMA,
            pltpu.SemaphoreType.DMA,
            pltpu.SemaphoreType.DMA,
            pltpu.SemaphoreType.DMA,
            pltpu.SemaphoreType.DMA,
            pltpu.SemaphoreType.DMA,
        ],
    )(_body)
    return run(inp1, idx32, tail_x, tail_i)
